# baseline (speedup 1.0000x reference)
import functools

import jax
import jax.numpy as jnp
from jax import lax
from jax.experimental import pallas as pl
from jax.experimental.pallas import tpu as pltpu
from jax.experimental.pallas import tpu_sc as plsc

_NW = 32
_L = 16
_LN2 = 0.6931471805599453


def _bilin(m):
    c0 = m[0, 0]
    c1 = m[1, 0] - m[0, 0]
    c2 = m[0, 1] - m[0, 0]
    c3 = m[1, 1] - m[1, 0] - m[0, 1] + m[0, 0]
    return [c0, c1, c2, c3]


def _rsqrt16(d2):
    i = lax.bitcast_convert_type(d2, jnp.int32)
    y = lax.bitcast_convert_type(jnp.int32(0x5F3759DF) - (i >> 1), jnp.float32)
    h = 0.5 * d2
    for _ in range(3):
        y = y * (1.5 - h * y * y)
    return y


def _ln16(b):
    i = lax.bitcast_convert_type(b, jnp.int32)
    ex = ((i >> 23) & 0xFF) - 127
    m = lax.bitcast_convert_type((i & 0x7FFFFF) | 0x3F800000, jnp.float32)
    r = (m - 1.0) / (m + 1.0)
    r2 = r * r
    p = 2.0 / 7.0 + r2 * 0.0
    p = 2.0 / 5.0 + r2 * p
    p = 2.0 / 3.0 + r2 * p
    lnm = r * (2.0 + r2 * p)
    return ex.astype(jnp.float32) * _LN2 + lnm


def _sc_pair_kernel(fx_hbm, fy_hbm, fz_hbm, sp_hbm, par_hbm,
                    ofx_hbm, ofy_hbm, ofz_hbm, ope_hbm,
                    xv, yv, zv, sv, pv, obuf):
    n = 4096
    wid = lax.axis_index("s") * 2 + lax.axis_index("c")
    base = wid * (n // _NW)

    pltpu.sync_copy(fx_hbm, xv)
    pltpu.sync_copy(fy_hbm, yv)
    pltpu.sync_copy(fz_hbm, zv)
    pltpu.sync_copy(sp_hbm, sv)
    pltpu.sync_copy(par_hbm, pv)

    p0 = pv[pl.ds(0, _L)]
    p1 = pv[pl.ds(_L, _L)]

    def par(k):
        v = p0[k] if k < _L else p1[k - _L]
        return jnp.full((_L,), v, jnp.float32)

    cell = [[par(3 * a + b) for b in range(3)] for a in range(3)]
    pbc = [par(9 + a) for a in range(3)]
    cut2 = par(12)
    sc = [par(13 + t) for t in range(4)]
    ec = [par(17 + t) for t in range(4)]
    ac = [par(21 + t) for t in range(4)]

    lane = lax.iota(jnp.int32, _L)

    def group_body(g, _):
        ob = base + g * _L
        ox = xv[pl.ds(ob, _L)]
        oy = yv[pl.ds(ob, _L)]
        oz = zv[pl.ds(ob, _L)]
        osp = sv[pl.ds(ob, _L)]
        own_id = ob + lane
        s0 = sc[0] + sc[1] * osp
        s1 = sc[2] + sc[3] * osp
        e0 = ec[0] + ec[1] * osp
        e1 = ec[2] + ec[3] * osp
        a0 = ac[0] + ac[1] * osp
        a1 = ac[2] + ac[3] * osp

        def j_body(jb, acc):
            afx, afy, afz, ape = acc
            jx = xv[pl.ds(jb * _L, _L)]
            jy = yv[pl.ds(jb * _L, _L)]
            jz = zv[pl.ds(jb * _L, _L)]
            js = sv[pl.ds(jb * _L, _L)]
            for l in range(_L):
                jglob = jb * _L + l
                df = []
                for ovec, jvec in ((ox, jx), (oy, jy), (oz, jz)):
                    dd = jnp.full((_L,), jvec[l], jnp.float32) - ovec
                    w = (jnp.where(dd > 0.5, 1.0, 0.0)
                         - jnp.where(dd < -0.5, 1.0, 0.0))
                    df.append(dd - w * pbc[len(df)])
                dr = [df[0] * cell[0][k] + df[1] * cell[1][k] + df[2] * cell[2][k]
                      for k in range(3)]
                d2 = dr[0] * dr[0] + dr[1] * dr[1] + dr[2] * dr[2]
                y = _rsqrt16(jnp.maximum(d2, 1e-12))
                d = d2 * y
                sj = jnp.full((_L,), js[l], jnp.float32)
                s = s0 + s1 * sj
                e = e0 + e1 * sj
                a = a0 + a1 * sj
                inside = (d2 < cut2) & (d < s) & (own_id != jglob)
                b = jnp.where(inside, 1.0 - d / s, 0.5)
                lnb = _ln16(b)
                p = jnp.exp(a * lnb)
                q = p / b
                pe = jnp.where(inside, (e / a) * p, 0.0)
                cf = jnp.where(inside, -(e / s) * q * y, 0.0)
                afx = afx + cf * dr[0]
                afy = afy + cf * dr[1]
                afz = afz + cf * dr[2]
                ape = ape + pe
            return afx, afy, afz, ape

        z = jnp.zeros((_L,), jnp.float32)
        afx, afy, afz, ape = lax.fori_loop(0, n // _L, j_body, (z, z, z, z))
        obuf[pl.ds(g * _L, _L)] = afx
        obuf[pl.ds(128 + g * _L, _L)] = afy
        obuf[pl.ds(256 + g * _L, _L)] = afz
        obuf[pl.ds(384 + g * _L, _L)] = ape
        return 0

    lax.fori_loop(0, (n // _NW) // _L, group_body, 0)

    pltpu.sync_copy(obuf.at[pl.ds(0, 128)], ofx_hbm.at[pl.ds(base, 128)])
    pltpu.sync_copy(obuf.at[pl.ds(128, 128)], ofy_hbm.at[pl.ds(base, 128)])
    pltpu.sync_copy(obuf.at[pl.ds(256, 128)], ofz_hbm.at[pl.ds(base, 128)])
    pltpu.sync_copy(obuf.at[pl.ds(384, 128)], ope_hbm.at[pl.ds(base, 128)])


def kernel(positions, cell, pbc, species, sigma_matrix, epsilon_matrix, alpha_matrix, cutoff):
    n = positions.shape[0]
    inv_cell = jnp.linalg.inv(cell)
    frac = positions @ inv_cell
    fx = frac[:, 0]
    fy = frac[:, 1]
    fz = frac[:, 2]
    sp = species.astype(jnp.float32)

    params = jnp.zeros((32,), jnp.float32)
    params = params.at[0:9].set(cell.reshape(9).astype(jnp.float32))
    params = params.at[9:12].set(pbc.astype(jnp.float32))
    cf32 = cutoff.astype(jnp.float32)
    params = params.at[12].set(cf32 * cf32)
    params = params.at[13:17].set(jnp.stack(_bilin(sigma_matrix)))
    params = params.at[17:21].set(jnp.stack(_bilin(epsilon_matrix)))
    params = params.at[21:25].set(jnp.stack(_bilin(alpha_matrix)))

    mesh = plsc.VectorSubcoreMesh(core_axis_name="c", subcore_axis_name="s")
    f = functools.partial(
        pl.kernel,
        mesh=mesh,
        out_type=[jax.ShapeDtypeStruct((n,), jnp.float32)] * 4,
        scratch_types=[
            pltpu.VMEM((n,), jnp.float32),
            pltpu.VMEM((n,), jnp.float32),
            pltpu.VMEM((n,), jnp.float32),
            pltpu.VMEM((n,), jnp.float32),
            pltpu.VMEM((32,), jnp.float32),
            pltpu.VMEM((512,), jnp.float32),
        ],
    )(_sc_pair_kernel)
    ofx, ofy, ofz, ope = f(fx, fy, fz, sp, params)

    forces = jnp.stack([ofx, ofy, ofz], axis=1)
    energy = 0.5 * jnp.sum(ope)
    return energy, forces


# speedup vs baseline: 5.9698x; 5.9698x over previous
import functools

import jax
import jax.numpy as jnp
from jax import lax
from jax.experimental import pallas as pl
from jax.experimental.pallas import tpu as pltpu
from jax.experimental.pallas import tpu_sc as plsc

_NW = 32
_L = 16
_LN2 = 0.6931471805599453
_R_SC = 512
_BI = 128


def _bilin(m):
    c0 = m[0, 0]
    c1 = m[1, 0] - m[0, 0]
    c2 = m[0, 1] - m[0, 0]
    c3 = m[1, 1] - m[1, 0] - m[0, 1] + m[0, 0]
    return jnp.stack([c0, c1, c2, c3])


def _rsqrt16(d2):
    i = lax.bitcast_convert_type(d2, jnp.int32)
    y = lax.bitcast_convert_type(jnp.int32(0x5F3759DF) - (i >> 1), jnp.float32)
    h = 0.5 * d2
    for _ in range(3):
        y = y * (1.5 - h * y * y)
    return y


def _ln16(b):
    i = lax.bitcast_convert_type(b, jnp.int32)
    ex = ((i >> 23) & 0xFF) - 127
    m = lax.bitcast_convert_type((i & 0x7FFFFF) | 0x3F800000, jnp.float32)
    r = (m - 1.0) / (m + 1.0)
    r2 = r * r
    p = 2.0 / 5.0 + r2 * (2.0 / 7.0)
    p = 2.0 / 3.0 + r2 * p
    lnm = r * (2.0 + r2 * p)
    return ex.astype(jnp.float32) * _LN2 + lnm


def _sc_pair_kernel(fx_hbm, fy_hbm, fz_hbm, sp_hbm, par_hbm,
                    ofx_hbm, ofy_hbm, ofz_hbm, ope_hbm,
                    xv, yv, zv, sv, pv, obuf):
    n = 4096
    rows = _R_SC // _NW
    wid = lax.axis_index("s") * 2 + lax.axis_index("c")
    base = wid * rows

    pltpu.sync_copy(fx_hbm, xv)
    pltpu.sync_copy(fy_hbm, yv)
    pltpu.sync_copy(fz_hbm, zv)
    pltpu.sync_copy(sp_hbm, sv)
    pltpu.sync_copy(par_hbm, pv)

    p0 = pv[pl.ds(0, _L)]
    p1 = pv[pl.ds(_L, _L)]

    def par(k):
        v = p0[k] if k < _L else p1[k - _L]
        return jnp.full((_L,), v, jnp.float32)

    cell = [[par(3 * a + b) for b in range(3)] for a in range(3)]
    pbc = [par(9 + a) for a in range(3)]
    cut2 = par(12)
    isc = [par(14 + t) for t in range(4)]
    acc_ = [par(18 + t) for t in range(4)]
    eoa = [par(22 + t) for t in range(4)]
    neos = [par(26 + t) for t in range(4)]

    lane = lax.iota(jnp.int32, _L)

    def group_body(g, _):
        ob = base + g * _L
        ox = xv[pl.ds(ob, _L)]
        oy = yv[pl.ds(ob, _L)]
        oz = zv[pl.ds(ob, _L)]
        osp = sv[pl.ds(ob, _L)]
        own_id = ob + lane
        is0 = isc[0] + isc[1] * osp
        is1 = isc[2] + isc[3] * osp
        a0 = acc_[0] + acc_[1] * osp
        a1 = acc_[2] + acc_[3] * osp
        ea0 = eoa[0] + eoa[1] * osp
        ea1 = eoa[2] + eoa[3] * osp
        es0 = neos[0] + neos[1] * osp
        es1 = neos[2] + neos[3] * osp

        def j_body(jb, acc):
            afx, afy, afz, ape = acc
            jx = xv[pl.ds(jb * _L, _L)]
            jy = yv[pl.ds(jb * _L, _L)]
            jz = zv[pl.ds(jb * _L, _L)]
            js = sv[pl.ds(jb * _L, _L)]
            for l in range(_L):
                jglob = jb * _L + l
                df = []
                for ovec, jvec in ((ox, jx), (oy, jy), (oz, jz)):
                    dd = jnp.full((_L,), jvec[l], jnp.float32) - ovec
                    w = (jnp.where(dd > 0.5, 1.0, 0.0)
                         - jnp.where(dd < -0.5, 1.0, 0.0))
                    df.append(dd - w * pbc[len(df)])
                dr = [df[0] * cell[0][k] + df[1] * cell[1][k] + df[2] * cell[2][k]
                      for k in range(3)]
                d2 = dr[0] * dr[0] + dr[1] * dr[1] + dr[2] * dr[2]
                y = _rsqrt16(jnp.maximum(d2, 1e-12))
                d = d2 * y
                sj = jnp.full((_L,), js[l], jnp.float32)
                inv_s = is0 + is1 * sj
                a = a0 + a1 * sj
                e_a = ea0 + ea1 * sj
                ne_s = es0 + es1 * sj
                braw = 1.0 - d * inv_s
                inside = (d2 < cut2) & (braw > 0.0) & (own_id != jglob)
                b = jnp.where(inside, braw, 0.5)
                lnb = _ln16(b)
                p = jnp.exp(a * lnb)
                q = p / b
                pe = jnp.where(inside, e_a * p, 0.0)
                cf = jnp.where(inside, ne_s * q * y, 0.0)
                afx = afx + cf * dr[0]
                afy = afy + cf * dr[1]
                afz = afz + cf * dr[2]
                ape = ape + pe
            return afx, afy, afz, ape

        z = jnp.zeros((_L,), jnp.float32)
        afx, afy, afz, ape = lax.fori_loop(0, n // _L, j_body, (z, z, z, z))
        obuf[pl.ds(g * _L, _L)] = afx
        obuf[pl.ds(rows + g * _L, _L)] = afy
        obuf[pl.ds(2 * rows + g * _L, _L)] = afz
        obuf[pl.ds(3 * rows + g * _L, _L)] = ape
        return 0

    lax.fori_loop(0, rows // _L, group_body, 0)

    pltpu.sync_copy(obuf.at[pl.ds(0, rows)], ofx_hbm.at[pl.ds(base, rows)])
    pltpu.sync_copy(obuf.at[pl.ds(rows, rows)], ofy_hbm.at[pl.ds(base, rows)])
    pltpu.sync_copy(obuf.at[pl.ds(2 * rows, rows)], ofz_hbm.at[pl.ds(base, rows)])
    pltpu.sync_copy(obuf.at[pl.ds(3 * rows, rows)], ope_hbm.at[pl.ds(base, rows)])


def _tc_pair_kernel(params_ref, row_ref, col_ref, out_ref):
    n = row_ref.shape[1]
    bi = col_ref.shape[0]
    pid = pl.program_id(0)

    cell = [[params_ref[3 * m + k] for k in range(3)] for m in range(3)]
    pbc = [params_ref[9 + m] for m in range(3)]
    cutoff = params_ref[13]
    isc = [params_ref[14 + t] for t in range(4)]
    ac = [params_ref[18 + t] for t in range(4)]
    eoa = [params_ref[22 + t] for t in range(4)]
    neos = [params_ref[26 + t] for t in range(4)]

    dfrac = []
    for m in range(3):
        fi = col_ref[:, m].reshape(bi, 1)
        fj = row_ref[m, :].reshape(1, n)
        df = fj - fi
        df = df - jnp.round(df) * pbc[m]
        dfrac.append(df)

    dr = []
    for k in range(3):
        acc = dfrac[0] * cell[0][k]
        acc = acc + dfrac[1] * cell[1][k]
        acc = acc + dfrac[2] * cell[2][k]
        dr.append(acc)
    d2 = dr[0] * dr[0] + dr[1] * dr[1] + dr[2] * dr[2]

    i_glob = _R_SC + pid * bi + jax.lax.broadcasted_iota(jnp.int32, (bi, n), 0)
    j_glob = jax.lax.broadcasted_iota(jnp.int32, (bi, n), 1)
    eye = i_glob == j_glob

    safe_d2 = jnp.where(eye, 1.0, d2)
    inv_d = lax.rsqrt(safe_d2)
    d = safe_d2 * inv_d

    si = col_ref[:, 3].reshape(bi, 1)
    sj = row_ref[3, :].reshape(1, n)
    sij = si * sj

    def bl(c):
        return c[0] + c[1] * si + c[2] * sj + c[3] * sij

    inv_s = bl(isc)
    a = bl(ac)
    e_a = bl(eoa)
    ne_s = bl(neos)

    braw = 1.0 - d * inv_s
    inside = (d < cutoff) & (braw > 0.0) & jnp.logical_not(eye)
    b = jnp.where(inside, braw, 0.5)
    lnb = jnp.log(b)
    q = jnp.exp((a - 1.0) * lnb)
    p = q * b

    pe = jnp.where(inside, e_a * p, 0.0)
    coeff = jnp.where(inside, ne_s * q * inv_d, 0.0)

    fx = jnp.sum(coeff * dr[0], axis=1).reshape(bi, 1)
    fy = jnp.sum(coeff * dr[1], axis=1).reshape(bi, 1)
    fz = jnp.sum(coeff * dr[2], axis=1).reshape(bi, 1)
    pes = jnp.sum(pe, axis=1).reshape(bi, 1)
    zeros = jnp.zeros((bi, 4), dtype=jnp.float32)
    out_ref[...] = jnp.concatenate([fx, fy, fz, pes, zeros], axis=1)


def kernel(positions, cell, pbc, species, sigma_matrix, epsilon_matrix, alpha_matrix, cutoff):
    n = positions.shape[0]
    inv_cell = jnp.linalg.inv(cell)
    frac = positions @ inv_cell
    spf = species.astype(jnp.float32)

    cf32 = cutoff.astype(jnp.float32)
    params = jnp.zeros((32,), jnp.float32)
    params = params.at[0:9].set(cell.reshape(9).astype(jnp.float32))
    params = params.at[9:12].set(pbc.astype(jnp.float32))
    params = params.at[12].set(cf32 * cf32)
    params = params.at[13].set(cf32)
    params = params.at[14:18].set(_bilin(1.0 / sigma_matrix))
    params = params.at[18:22].set(_bilin(alpha_matrix))
    params = params.at[22:26].set(_bilin(epsilon_matrix / alpha_matrix))
    params = params.at[26:30].set(_bilin(-epsilon_matrix / sigma_matrix))

    mesh = plsc.VectorSubcoreMesh(core_axis_name="c", subcore_axis_name="s")
    sc_f = functools.partial(
        pl.kernel,
        mesh=mesh,
        out_type=[jax.ShapeDtypeStruct((_R_SC,), jnp.float32)] * 4,
        scratch_types=[
            pltpu.VMEM((n,), jnp.float32),
            pltpu.VMEM((n,), jnp.float32),
            pltpu.VMEM((n,), jnp.float32),
            pltpu.VMEM((n,), jnp.float32),
            pltpu.VMEM((32,), jnp.float32),
            pltpu.VMEM((4 * _R_SC // _NW,), jnp.float32),
        ],
    )(_sc_pair_kernel)
    ofx, ofy, ofz, ope = sc_f(frac[:, 0], frac[:, 1], frac[:, 2], spf, params)

    col = jnp.concatenate(
        [frac, spf[:, None], jnp.zeros((n, 4), jnp.float32)], axis=1)
    row = col.T

    grid = ((n - _R_SC) // _BI,)
    tc_out = pl.pallas_call(
        _tc_pair_kernel,
        grid=grid,
        in_specs=[
            pl.BlockSpec(memory_space=pltpu.SMEM),
            pl.BlockSpec((8, n), lambda i: (0, 0)),
            pl.BlockSpec((_BI, 8), lambda i: (i + _R_SC // _BI, 0)),
        ],
        out_specs=pl.BlockSpec((_BI, 8), lambda i: (i, 0)),
        out_shape=jax.ShapeDtypeStruct((n - _R_SC, 8), jnp.float32),
    )(params, row, col)

    forces = jnp.concatenate(
        [jnp.stack([ofx, ofy, ofz], axis=1), tc_out[:, :3]], axis=0)
    energy = 0.5 * (jnp.sum(ope) + jnp.sum(tc_out[:, 3]))
    return energy, forces
